# baseline (device time: 239175 ns/iter reference)
import jax
import jax.numpy as jnp
from jax import lax
from jax.experimental import pallas as pl
from jax.experimental.pallas import tpu as pltpu

N_DEV = 4
HQ_LOC = 8
HQ = N_DEV * HQ_LOC
DH = 128
SQ = 2048
SKV = 2048
QC = 512
CH = SQ // N_DEV
HC = 512
SCALE = 0.08838834764831843
NEG = -1e9
SEND_SLOTS = 14
STAG_SLOTS = 3
DEPTH = 2
NFWD = HQ_LOC // 2
MESH = pl.DeviceIdType.MESH


def _body(xb_ref, wq_ref, k_hbm, v_hbm, wo_ref, out_ref,
          kbuf, vbuf, kpool, vpool, stag, fwdk, fwdv,
          pb, rs_stage, ag_stage, comm_rs, comm_ag,
          recv_k, recv_v, send_k, send_v, stag_sem,
          fwd_recv_k, fwd_recv_v, fwd_send_k, fwd_send_v,
          rs_send, rs_recv, ag_send, ag_recv):
    my = lax.axis_index("i")

    @pl.when(my == 0)
    def _():
        tasks = []
        for h in range(HQ_LOC):
            for kind in ("f", 1, 3, 0):
                tasks.append((0, kind, h))
                tasks.append((1, kind, h))

        def stag_copy(i):
            is_v, kind, h = tasks[i]
            g = (2 if kind == "f" else kind) * HQ_LOC + h
            src = (v_hbm if is_v else k_hbm).at[:, g, :]
            return pltpu.make_async_copy(
                src, stag.at[i % STAG_SLOTS], stag_sem.at[i % STAG_SLOTS])

        for i in range(DEPTH):
            stag_copy(i).start()

        nsend = [0, 0]
        for i, t in enumerate(tasks):
            if i + DEPTH < len(tasks):
                stag_copy(i + DEPTH).start()
            stag_copy(i).wait()
            is_v, kind, h = t
            data = stag[i % STAG_SLOTS].astype(jnp.bfloat16)
            if kind == 0:
                (vbuf if is_v else kbuf)[h, :, :] = data
                continue
            pool = vpool if is_v else kpool
            sems = send_v if is_v else send_k
            slot = nsend[is_v] % SEND_SLOTS
            if nsend[is_v] >= SEND_SLOTS:
                pltpu.make_async_remote_copy(
                    src_ref=pool.at[slot], dst_ref=kbuf.at[0],
                    send_sem=sems.at[slot], recv_sem=recv_k.at[0],
                    device_id=(1,), device_id_type=MESH,
                ).wait_send()
            pool[slot, :, :] = data
            if kind == "f":
                dest = 1 if h < NFWD else 3
                dst = (fwdv if is_v else fwdk).at[h % NFWD]
                rsem = (fwd_recv_v if is_v else fwd_recv_k).at[h % NFWD]
            else:
                dest = kind
                dst = (vbuf if is_v else kbuf).at[h]
                rsem = (recv_v if is_v else recv_k).at[h]
            pltpu.make_async_remote_copy(
                src_ref=pool.at[slot], dst_ref=dst,
                send_sem=sems.at[slot], recv_sem=rsem,
                device_id=(dest,), device_id_type=MESH,
            ).start()
            nsend[is_v] += 1

    for h in range(HQ_LOC):
        fwd_owner = 1 if h < NFWD else 3
        s = h % NFWD

        @pl.when(my == fwd_owner)
        def _():
            for buf, ssem, rsem, dsem in (
                (fwdk, fwd_send_k, fwd_recv_k, recv_k),
                (fwdv, fwd_send_v, fwd_recv_v, recv_v),
            ):
                pltpu.make_async_remote_copy(
                    src_ref=buf.at[s], dst_ref=buf.at[s],
                    send_sem=ssem.at[s], recv_sem=rsem.at[s],
                    device_id=(0,), device_id_type=MESH,
                ).wait_recv()
                dst = kbuf if buf is fwdk else vbuf
                pltpu.make_async_remote_copy(
                    src_ref=buf.at[s], dst_ref=dst.at[h],
                    send_sem=ssem.at[s], recv_sem=dsem.at[h],
                    device_id=(2,), device_id_type=MESH,
                ).start()

        @pl.when(my != 0)
        def _():
            pltpu.make_async_remote_copy(
                src_ref=kbuf.at[h], dst_ref=kbuf.at[h],
                send_sem=send_k.at[0], recv_sem=recv_k.at[h],
                device_id=(0,), device_id_type=MESH,
            ).wait_recv()
            pltpu.make_async_remote_copy(
                src_ref=vbuf.at[h], dst_ref=vbuf.at[h],
                send_sem=send_v.at[0], recv_sem=recv_v.at[h],
                device_id=(0,), device_id_type=MESH,
            ).wait_recv()

        qh = jnp.dot(xb_ref[:, :], wq_ref[:, h * DH:(h + 1) * DH],
                     preferred_element_type=jnp.float32)
        qb = qh.astype(jnp.bfloat16)
        wo_h = wo_ref[h * DH:(h + 1) * DH, :]

        for qc in range(SQ // QC):
            w = (qc + 1) * QC
            sc = lax.dot_general(
                qb[qc * QC:(qc + 1) * QC, :], kbuf[h, :w, :],
                (((1,), (1,)), ((), ())),
                preferred_element_type=jnp.float32,
            ) * SCALE
            iblk = (lax.broadcasted_iota(jnp.int32, (QC, w), 0) // 64
                    + qc * (QC // 64))
            jblk = lax.broadcasted_iota(jnp.int32, (QC, w), 1) // 64
            sc = jnp.where(jblk <= iblk, sc, NEG)
            mx = jnp.max(sc, axis=1, keepdims=True)
            e = jnp.exp(sc - mx)
            p = (e / jnp.sum(e, axis=1, keepdims=True)).astype(jnp.bfloat16)
            ctx = jnp.dot(p, vbuf[h, :w, :],
                          preferred_element_type=jnp.float32)
            contrib = jnp.dot(ctx.astype(jnp.bfloat16), wo_h,
                              preferred_element_type=jnp.float32)
            row = pl.ds(qc * QC, QC)
            if h == 0:
                out_ref[row, :] = contrib
            else:
                out_ref[row, :] = out_ref[row, :] + contrib

    @pl.when(my == 0)
    def _():
        for slot in range(SEND_SLOTS):
            pltpu.make_async_remote_copy(
                src_ref=kpool.at[slot], dst_ref=kbuf.at[0],
                send_sem=send_k.at[slot], recv_sem=recv_k.at[0],
                device_id=(1,), device_id_type=MESH,
            ).wait_send()
            pltpu.make_async_remote_copy(
                src_ref=vpool.at[slot], dst_ref=vbuf.at[0],
                send_sem=send_v.at[slot], recv_sem=recv_v.at[0],
                device_id=(1,), device_id_type=MESH,
            ).wait_send()

    @pl.when((my == 1) | (my == 3))
    def _():
        for s in range(NFWD):
            for buf, ssem in ((fwdk, fwd_send_k), (fwdv, fwd_send_v)):
                pltpu.make_async_remote_copy(
                    src_ref=buf.at[s], dst_ref=buf.at[s],
                    send_sem=ssem.at[s], recv_sem=fwd_recv_k.at[s],
                    device_id=(2,), device_id_type=MESH,
                ).wait_send()

    def rows(c):
        return pl.ds(c * CH, CH)

    def cols(d):
        return pl.ds(d * HC, HC)

    pb[:, :] = out_ref[pl.ds(my * CH, CH), :].astype(jnp.bfloat16)
    tgt = [lax.rem(my + 1, N_DEV), lax.rem(my + 3, N_DEV)]

    for k in range(N_DEV - 1):
        rdmas = []
        for d in (0, 1):
            src = pb.at[:, cols(d)] if k == 0 else rs_stage.at[d, k - 1]
            rdma = pltpu.make_async_remote_copy(
                src_ref=src, dst_ref=comm_rs.at[d, k],
                send_sem=rs_send.at[d, k], recv_sem=rs_recv.at[d, k],
                device_id=(tgt[d],), device_id_type=MESH,
            )
            rdma.start()
            rdmas.append(rdma)
        for d in (0, 1):
            rdmas[d].wait()
            c = lax.rem(my + (N_DEV - 1 - k if d == 0 else 1 + k), N_DEV)
            acc = comm_rs[d, k].astype(jnp.float32) + out_ref[rows(c), cols(d)]
            if k < N_DEV - 2:
                rs_stage[d, k, :, :] = acc.astype(jnp.bfloat16)
            else:
                out_ref[rows(c), cols(d)] = acc
                ag_stage[d, :, :] = acc.astype(jnp.bfloat16)

    for k in range(N_DEV - 1):
        rdmas = []
        for d in (0, 1):
            src = ag_stage.at[d] if k == 0 else comm_ag.at[d, k - 1]
            rdma = pltpu.make_async_remote_copy(
                src_ref=src, dst_ref=comm_ag.at[d, k],
                send_sem=ag_send.at[d, k], recv_sem=ag_recv.at[d, k],
                device_id=(tgt[d],), device_id_type=MESH,
            )
            rdma.start()
            rdmas.append(rdma)
        for d in (0, 1):
            rdmas[d].wait()
            c = lax.rem(my + (N_DEV - k if d == 0 else k), N_DEV)
            out_ref[rows(c), cols(d)] = comm_ag[d, k].astype(jnp.float32)


def kernel(x, Wq, K_ext, V_ext, Wo):
    xb = x.reshape(SQ, 1024).astype(jnp.bfloat16)
    wqb = Wq.astype(jnp.bfloat16)
    wob = Wo.astype(jnp.bfloat16)
    k = K_ext.reshape(SKV, HQ, DH)
    v = V_ext.reshape(SKV, HQ, DH)

    out = pl.pallas_call(
        _body,
        out_shape=jax.ShapeDtypeStruct((SQ, 1024), jnp.float32),
        in_specs=[
            pl.BlockSpec(memory_space=pltpu.VMEM),
            pl.BlockSpec(memory_space=pltpu.VMEM),
            pl.BlockSpec(memory_space=pl.ANY),
            pl.BlockSpec(memory_space=pl.ANY),
            pl.BlockSpec(memory_space=pltpu.VMEM),
        ],
        out_specs=pl.BlockSpec(memory_space=pltpu.VMEM),
        scratch_shapes=[
            pltpu.VMEM((HQ_LOC, SKV, DH), jnp.bfloat16),
            pltpu.VMEM((HQ_LOC, SKV, DH), jnp.bfloat16),
            pltpu.VMEM((SEND_SLOTS, SKV, DH), jnp.bfloat16),
            pltpu.VMEM((SEND_SLOTS, SKV, DH), jnp.bfloat16),
            pltpu.VMEM((STAG_SLOTS, SKV, DH), jnp.float32),
            pltpu.VMEM((NFWD, SKV, DH), jnp.bfloat16),
            pltpu.VMEM((NFWD, SKV, DH), jnp.bfloat16),
            pltpu.VMEM((CH, 2 * HC), jnp.bfloat16),
            pltpu.VMEM((2, N_DEV - 2, CH, HC), jnp.bfloat16),
            pltpu.VMEM((2, CH, HC), jnp.bfloat16),
            pltpu.VMEM((2, N_DEV - 1, CH, HC), jnp.bfloat16),
            pltpu.VMEM((2, N_DEV - 1, CH, HC), jnp.bfloat16),
            pltpu.SemaphoreType.DMA((HQ_LOC,)),
            pltpu.SemaphoreType.DMA((HQ_LOC,)),
            pltpu.SemaphoreType.DMA((SEND_SLOTS,)),
            pltpu.SemaphoreType.DMA((SEND_SLOTS,)),
            pltpu.SemaphoreType.DMA((STAG_SLOTS,)),
            pltpu.SemaphoreType.DMA((NFWD,)),
            pltpu.SemaphoreType.DMA((NFWD,)),
            pltpu.SemaphoreType.DMA((NFWD,)),
            pltpu.SemaphoreType.DMA((NFWD,)),
            pltpu.SemaphoreType.DMA((2, N_DEV - 1)),
            pltpu.SemaphoreType.DMA((2, N_DEV - 1)),
            pltpu.SemaphoreType.DMA((2, N_DEV - 1)),
            pltpu.SemaphoreType.DMA((2, N_DEV - 1)),
        ],
        compiler_params=pltpu.CompilerParams(
            vmem_limit_bytes=62 * 1024 * 1024,
        ),
    )(xb, wqb, k, v, wob)

    return out.reshape(1, SQ, 1024)


# device time: 223665 ns/iter; 1.0693x vs baseline; 1.0693x over previous
import jax
import jax.numpy as jnp
from jax import lax
from jax.experimental import pallas as pl
from jax.experimental.pallas import tpu as pltpu

N_DEV = 4
HQ_LOC = 8
HQ = N_DEV * HQ_LOC
DH = 128
SQ = 2048
SKV = 2048
QC = 512
CH = SQ // N_DEV
HC = 512
SCALE = 0.08838834764831843
NEG = -1e9
SEND_SLOTS = 16
STAG_SLOTS = 4
DEPTH = 3
NFWD = HQ_LOC // 2
MESH = pl.DeviceIdType.MESH


def _attn_body(xb_ref, wq_ref, k_hbm, v_hbm, wo_ref, out_ref,
               kbuf, vbuf, kpool, vpool, stag, fwdk, fwdv,
               recv_k, recv_v, send_k, send_v, stag_sem,
               fwd_recv_k, fwd_recv_v, fwd_send_k, fwd_send_v):
    my = lax.axis_index("i")

    @pl.when(my == 0)
    def _():
        tasks = []
        for h in range(HQ_LOC):
            for kind in ("f", 1, 3, 0):
                tasks.append((0, kind, h))
                tasks.append((1, kind, h))

        def stag_copy(i):
            is_v, kind, h = tasks[i]
            g = (2 if kind == "f" else kind) * HQ_LOC + h
            src = (v_hbm if is_v else k_hbm).at[:, g, :]
            return pltpu.make_async_copy(
                src, stag.at[i % STAG_SLOTS], stag_sem.at[i % STAG_SLOTS])

        for i in range(DEPTH):
            stag_copy(i).start()

        nsend = [0, 0]
        for i, t in enumerate(tasks):
            if i + DEPTH < len(tasks):
                stag_copy(i + DEPTH).start()
            stag_copy(i).wait()
            is_v, kind, h = t
            data = stag[i % STAG_SLOTS].astype(jnp.bfloat16)
            if kind == 0:
                (vbuf if is_v else kbuf)[h, :, :] = data
                continue
            pool = vpool if is_v else kpool
            sems = send_v if is_v else send_k
            slot = nsend[is_v] % SEND_SLOTS
            if nsend[is_v] >= SEND_SLOTS:
                pltpu.make_async_remote_copy(
                    src_ref=pool.at[slot], dst_ref=kbuf.at[0],
                    send_sem=sems.at[slot], recv_sem=recv_k.at[0],
                    device_id=(1,), device_id_type=MESH,
                ).wait_send()
            pool[slot, :, :] = data
            if kind == "f":
                dest = 1 if h < NFWD else 3
                dst = (fwdv if is_v else fwdk).at[h % NFWD]
                rsem = (fwd_recv_v if is_v else fwd_recv_k).at[h % NFWD]
            else:
                dest = kind
                dst = (vbuf if is_v else kbuf).at[h]
                rsem = (recv_v if is_v else recv_k).at[h]
            pltpu.make_async_remote_copy(
                src_ref=pool.at[slot], dst_ref=dst,
                send_sem=sems.at[slot], recv_sem=rsem,
                device_id=(dest,), device_id_type=MESH,
            ).start()
            nsend[is_v] += 1

    for h in range(HQ_LOC):
        fwd_owner = 1 if h < NFWD else 3
        s = h % NFWD

        @pl.when(my == fwd_owner)
        def _():
            for buf, ssem, rsem, dsem in (
                (fwdk, fwd_send_k, fwd_recv_k, recv_k),
                (fwdv, fwd_send_v, fwd_recv_v, recv_v),
            ):
                pltpu.make_async_remote_copy(
                    src_ref=buf.at[s], dst_ref=buf.at[s],
                    send_sem=ssem.at[s], recv_sem=rsem.at[s],
                    device_id=(0,), device_id_type=MESH,
                ).wait_recv()
                dst = kbuf if buf is fwdk else vbuf
                pltpu.make_async_remote_copy(
                    src_ref=buf.at[s], dst_ref=dst.at[h],
                    send_sem=ssem.at[s], recv_sem=dsem.at[h],
                    device_id=(2,), device_id_type=MESH,
                ).start()

        @pl.when(my != 0)
        def _():
            pltpu.make_async_remote_copy(
                src_ref=kbuf.at[h], dst_ref=kbuf.at[h],
                send_sem=send_k.at[0], recv_sem=recv_k.at[h],
                device_id=(0,), device_id_type=MESH,
            ).wait_recv()
            pltpu.make_async_remote_copy(
                src_ref=vbuf.at[h], dst_ref=vbuf.at[h],
                send_sem=send_v.at[0], recv_sem=recv_v.at[h],
                device_id=(0,), device_id_type=MESH,
            ).wait_recv()

        qh = jnp.dot(xb_ref[:, :], wq_ref[:, h * DH:(h + 1) * DH],
                     preferred_element_type=jnp.float32)
        qb = qh.astype(jnp.bfloat16)
        wo_h = wo_ref[h * DH:(h + 1) * DH, :]

        for qc in range(SQ // QC):
            w = (qc + 1) * QC
            sc = lax.dot_general(
                qb[qc * QC:(qc + 1) * QC, :], kbuf[h, :w, :],
                (((1,), (1,)), ((), ())),
                preferred_element_type=jnp.float32,
            ) * SCALE
            iblk = (lax.broadcasted_iota(jnp.int32, (QC, w), 0) // 64
                    + qc * (QC // 64))
            jblk = lax.broadcasted_iota(jnp.int32, (QC, w), 1) // 64
            sc = jnp.where(jblk <= iblk, sc, NEG)
            mx = jnp.max(sc, axis=1, keepdims=True)
            e = jnp.exp(sc - mx)
            p = (e / jnp.sum(e, axis=1, keepdims=True)).astype(jnp.bfloat16)
            ctx = jnp.dot(p, vbuf[h, :w, :],
                          preferred_element_type=jnp.float32)
            contrib = jnp.dot(ctx.astype(jnp.bfloat16), wo_h,
                              preferred_element_type=jnp.float32)
            row = pl.ds(qc * QC, QC)
            if h == 0:
                out_ref[row, :] = contrib
            else:
                out_ref[row, :] = out_ref[row, :] + contrib

    @pl.when(my == 0)
    def _():
        for slot in range(SEND_SLOTS):
            pltpu.make_async_remote_copy(
                src_ref=kpool.at[slot], dst_ref=kbuf.at[0],
                send_sem=send_k.at[slot], recv_sem=recv_k.at[0],
                device_id=(1,), device_id_type=MESH,
            ).wait_send()
            pltpu.make_async_remote_copy(
                src_ref=vpool.at[slot], dst_ref=vbuf.at[0],
                send_sem=send_v.at[slot], recv_sem=recv_v.at[0],
                device_id=(1,), device_id_type=MESH,
            ).wait_send()

    @pl.when((my == 1) | (my == 3))
    def _():
        for s in range(NFWD):
            for buf, ssem in ((fwdk, fwd_send_k), (fwdv, fwd_send_v)):
                pltpu.make_async_remote_copy(
                    src_ref=buf.at[s], dst_ref=buf.at[s],
                    send_sem=ssem.at[s], recv_sem=fwd_recv_k.at[s],
                    device_id=(2,), device_id_type=MESH,
                ).wait_send()


def _ar_body(in_ref, out_ref, pb, rs_stage, ag_stage, comm_rs, comm_ag,
             rs_send, rs_recv, ag_send, ag_recv):
    my = lax.axis_index("i")

    def rows(c):
        return pl.ds(c * CH, CH)

    def cols(d):
        return pl.ds(d * HC, HC)

    pb[:, :] = in_ref[:, :].astype(jnp.bfloat16)
    tgt = [lax.rem(my + 1, N_DEV), lax.rem(my + 3, N_DEV)]

    for k in range(N_DEV - 1):
        rdmas = []
        for d in (0, 1):
            src = (pb.at[rows(my), cols(d)] if k == 0
                   else rs_stage.at[d, k - 1])
            rdma = pltpu.make_async_remote_copy(
                src_ref=src, dst_ref=comm_rs.at[d, k],
                send_sem=rs_send.at[d, k], recv_sem=rs_recv.at[d, k],
                device_id=(tgt[d],), device_id_type=MESH,
            )
            rdma.start()
            rdmas.append(rdma)
        for d in (0, 1):
            rdmas[d].wait()
            c = lax.rem(my + (N_DEV - 1 - k if d == 0 else 1 + k), N_DEV)
            acc = comm_rs[d, k].astype(jnp.float32) + in_ref[rows(c), cols(d)]
            if k < N_DEV - 2:
                rs_stage[d, k, :, :] = acc.astype(jnp.bfloat16)
            else:
                out_ref[rows(c), cols(d)] = acc
                ag_stage[d, :, :] = acc.astype(jnp.bfloat16)

    for k in range(N_DEV - 1):
        rdmas = []
        for d in (0, 1):
            src = ag_stage.at[d] if k == 0 else comm_ag.at[d, k - 1]
            rdma = pltpu.make_async_remote_copy(
                src_ref=src, dst_ref=comm_ag.at[d, k],
                send_sem=ag_send.at[d, k], recv_sem=ag_recv.at[d, k],
                device_id=(tgt[d],), device_id_type=MESH,
            )
            rdma.start()
            rdmas.append(rdma)
        for d in (0, 1):
            rdmas[d].wait()
            c = lax.rem(my + (N_DEV - k if d == 0 else k), N_DEV)
            out_ref[rows(c), cols(d)] = comm_ag[d, k].astype(jnp.float32)


def kernel(x, Wq, K_ext, V_ext, Wo):
    xb = x.reshape(SQ, 1024).astype(jnp.bfloat16)
    wqb = Wq.astype(jnp.bfloat16)
    wob = Wo.astype(jnp.bfloat16)
    k = K_ext.reshape(SKV, HQ, DH)
    v = V_ext.reshape(SKV, HQ, DH)

    partial = pl.pallas_call(
        _attn_body,
        out_shape=jax.ShapeDtypeStruct((SQ, 1024), jnp.float32),
        in_specs=[
            pl.BlockSpec(memory_space=pltpu.VMEM),
            pl.BlockSpec(memory_space=pltpu.VMEM),
            pl.BlockSpec(memory_space=pl.ANY),
            pl.BlockSpec(memory_space=pl.ANY),
            pl.BlockSpec(memory_space=pltpu.VMEM),
        ],
        out_specs=pl.BlockSpec(memory_space=pltpu.VMEM),
        scratch_shapes=[
            pltpu.VMEM((HQ_LOC, SKV, DH), jnp.bfloat16),
            pltpu.VMEM((HQ_LOC, SKV, DH), jnp.bfloat16),
            pltpu.VMEM((SEND_SLOTS, SKV, DH), jnp.bfloat16),
            pltpu.VMEM((SEND_SLOTS, SKV, DH), jnp.bfloat16),
            pltpu.VMEM((STAG_SLOTS, SKV, DH), jnp.float32),
            pltpu.VMEM((NFWD, SKV, DH), jnp.bfloat16),
            pltpu.VMEM((NFWD, SKV, DH), jnp.bfloat16),
            pltpu.SemaphoreType.DMA((HQ_LOC,)),
            pltpu.SemaphoreType.DMA((HQ_LOC,)),
            pltpu.SemaphoreType.DMA((SEND_SLOTS,)),
            pltpu.SemaphoreType.DMA((SEND_SLOTS,)),
            pltpu.SemaphoreType.DMA((STAG_SLOTS,)),
            pltpu.SemaphoreType.DMA((NFWD,)),
            pltpu.SemaphoreType.DMA((NFWD,)),
            pltpu.SemaphoreType.DMA((NFWD,)),
            pltpu.SemaphoreType.DMA((NFWD,)),
        ],
        compiler_params=pltpu.CompilerParams(
            vmem_limit_bytes=60 * 1024 * 1024,
            skip_device_barrier=True,
        ),
    )(xb, wqb, k, v, wob)

    out = pl.pallas_call(
        _ar_body,
        out_shape=jax.ShapeDtypeStruct((SQ, 1024), jnp.float32),
        in_specs=[pl.BlockSpec(memory_space=pltpu.VMEM)],
        out_specs=pl.BlockSpec(memory_space=pltpu.VMEM),
        scratch_shapes=[
            pltpu.VMEM((SQ, 1024), jnp.bfloat16),
            pltpu.VMEM((2, N_DEV - 2, CH, HC), jnp.bfloat16),
            pltpu.VMEM((2, CH, HC), jnp.bfloat16),
            pltpu.VMEM((2, N_DEV - 1, CH, HC), jnp.bfloat16),
            pltpu.VMEM((2, N_DEV - 1, CH, HC), jnp.bfloat16),
            pltpu.SemaphoreType.DMA((2, N_DEV - 1)),
            pltpu.SemaphoreType.DMA((2, N_DEV - 1)),
            pltpu.SemaphoreType.DMA((2, N_DEV - 1)),
            pltpu.SemaphoreType.DMA((2, N_DEV - 1)),
        ],
        compiler_params=pltpu.CompilerParams(
            skip_device_barrier=True,
        ),
    )(partial)

    return out.reshape(1, SQ, 1024)
